# Initial kernel scaffold; baseline (speedup 1.0000x reference)
import jax
import jax.numpy as jnp
from jax.experimental import pallas as pl


def kernel(h, targets, Wg_mfs, bg_mfs, Wf, bf, Wg_e, bg_e, Ws, gamma, beta, Wc, bc):
    raise NotImplementedError("write your pallas kernel here")


# speedup vs baseline: 8.3919x; 8.3919x over previous
import jax
import jax.numpy as jnp
from jax.experimental import pallas as pl
from jax.experimental.pallas import tpu as pltpu

_S_THRESH = 0.7
_BOOST_GAIN = 2.0
_PAIR_WEIGHT = 0.5
_EPS = 1e-9


def _aha_last_token_kernel(t_ref, h_ref, wg_mfs_ref, bg_mfs_ref, wf_ref,
                           bf_ref, wg_e_ref, bg_e_ref, ws_ref, gamma_ref,
                           beta_ref, wc_ref, bc_ref, out_ref):
    K = wf_ref.shape[0]
    V = wf_ref.shape[2]

    t = t_ref[0, 0]
    hv = h_ref[...]

    g_log = jnp.dot(hv, wg_mfs_ref[...],
                    preferred_element_type=jnp.float32) + bg_mfs_ref[...]
    g = jax.nn.softmax(g_log, axis=-1)
    logg = jnp.log(jnp.clip(g, _EPS, None))

    vocab_iota = jax.lax.broadcasted_iota(jnp.int32, (1, V), 1)
    t_mask = vocab_iota == t

    lp_parts = []
    st_parts = []
    for k in range(K):
        logits = jnp.dot(hv, wf_ref[k],
                         preferred_element_type=jnp.float32) + bf_ref[k:k + 1, :]
        m = jnp.max(logits, axis=-1, keepdims=True)
        lse = m + jnp.log(jnp.sum(jnp.exp(logits - m), axis=-1, keepdims=True))
        val = jnp.sum(jnp.where(t_mask, logits, 0.0), axis=-1, keepdims=True)
        lp_parts.append(val - lse)
        st_parts.append(jnp.dot(hv, ws_ref[k],
                                preferred_element_type=jnp.float32))
    logp = jnp.concatenate(lp_parts, axis=-1)
    states = jnp.concatenate(st_parts, axis=0)

    mix_in = logg + logp
    mm = jnp.max(mix_in, axis=-1, keepdims=True)
    log_mix = mm + jnp.log(jnp.sum(jnp.exp(mix_in - mm), axis=-1,
                                   keepdims=True))
    s = logp - log_mix

    G_log = jnp.dot(hv, wg_e_ref[...],
                    preferred_element_type=jnp.float32) + bg_e_ref[...]
    G = jax.nn.softmax(G_log, axis=-1)
    k_iota = jax.lax.broadcasted_iota(jnp.int32, (1, K), 1)
    m1 = jnp.max(G, axis=-1, keepdims=True)
    i1 = jnp.min(jnp.where(G == m1, k_iota, K), axis=-1, keepdims=True)
    oh1 = k_iota == i1
    G_rem = jnp.where(oh1, -1.0, G)
    m2 = jnp.max(G_rem, axis=-1, keepdims=True)
    i2 = jnp.min(jnp.where(G_rem == m2, k_iota, K), axis=-1, keepdims=True)
    sel_mask = oh1 | (k_iota == i2)

    leftover = G * (1.0 - sel_mask.astype(jnp.float32))
    aha = (s > _S_THRESH) & (~sel_mask)
    boosted = leftover * jnp.where(aha, _BOOST_GAIN, 1.0)
    any_aha = jnp.any(aha, axis=-1, keepdims=True)
    boosted = jnp.where(any_aha,
                        boosted + _PAIR_WEIGHT * oh1.astype(jnp.float32),
                        boosted)
    boosted = boosted / jnp.clip(jnp.sum(boosted, axis=-1, keepdims=True),
                                 1e-9, None)

    b = jnp.dot(boosted, states, preferred_element_type=jnp.float32)
    mu = jnp.mean(b, axis=-1, keepdims=True)
    d = b - mu
    var = jnp.mean(d * d, axis=-1, keepdims=True)
    bn = d * jax.lax.rsqrt(var + 1e-5) * gamma_ref[...] + beta_ref[...]
    out_ref[...] = jnp.dot(bn, wc_ref[...],
                           preferred_element_type=jnp.float32) + bc_ref[...]


def kernel(h, targets, Wg_mfs, bg_mfs, Wf, bf, Wg_e, bg_e, Ws, gamma, beta,
           Wc, bc):
    B, T, D = h.shape
    K, _, V = Wf.shape
    SD = Ws.shape[2]

    t_last = targets[:, -1:].astype(jnp.int32)
    h_last = h[:, -1, :]

    full = lambda shape: pl.BlockSpec(shape, lambda: (0,) * len(shape))
    out = pl.pallas_call(
        _aha_last_token_kernel,
        out_shape=jax.ShapeDtypeStruct((B, D), jnp.float32),
        in_specs=[
            pl.BlockSpec(memory_space=pltpu.SMEM),
            full((1, D)),
            full((D, K)),
            full((1, K)),
            full((K, D, V)),
            full((K, V)),
            full((D, K)),
            full((1, K)),
            full((K, D, SD)),
            full((1, SD)),
            full((1, SD)),
            full((SD, D)),
            full((1, D)),
        ],
        out_specs=full((B, D)),
    )(t_last, h_last, Wg_mfs, bg_mfs.reshape(1, K), Wf, bf, Wg_e,
      bg_e.reshape(1, K), Ws, gamma.reshape(1, SD), beta.reshape(1, SD),
      Wc, bc.reshape(1, D))
    return out
